# baseline (speedup 1.0000x reference)
import functools

import jax
import jax.numpy as jnp
from jax import lax
from jax.experimental import pallas as pl
from jax.experimental.pallas import tpu as pltpu
from jax.experimental.pallas import tpu_sc as plsc

N = 10000
F = 128
E = 320000
NC = 2
NS = 16
NW = NC * NS
EPW = E // NW
CB = 80
NCHUNK = EPW // CB
G = 5
NG = NCHUNK // G
NBUF = 4
NFULL = ((G - 1) // NBUF) * NBUF
RPS = 624
TAIL0 = RPS * NS
TAILN = N - TAIL0

RB = 1000


def _mm1_body(x_ref, w_ref, o_ref):
    o_ref[...] = jnp.dot(x_ref[...], w_ref[...],
                         preferred_element_type=jnp.float32)


def _mm2_body(a_ref, w_ref, o_ref):
    h = jnp.maximum(a_ref[0] + a_ref[1], 0.0)
    o_ref[...] = jnp.dot(h, w_ref[...], preferred_element_type=jnp.float32)


def _lsm_body(a_ref, o_ref):
    s = a_ref[0] + a_ref[1]
    m = jnp.max(s, axis=-1, keepdims=True)
    e = jnp.exp(s - m)
    o_ref[...] = s - m - jnp.log(jnp.sum(e, axis=-1, keepdims=True))


def _matmul1(x, W):
    return pl.pallas_call(
        _mm1_body,
        grid=(N // RB,),
        in_specs=[pl.BlockSpec((RB, F), lambda i: (i, 0)),
                  pl.BlockSpec((F, F), lambda i: (0, 0))],
        out_specs=pl.BlockSpec((RB, F), lambda i: (i, 0)),
        out_shape=jax.ShapeDtypeStruct((N, F), jnp.float32),
    )(x, W)


def _relu_matmul2(acc, W):
    return pl.pallas_call(
        _mm2_body,
        grid=(N // RB,),
        in_specs=[pl.BlockSpec((NC, RB, F), lambda i: (0, i, 0)),
                  pl.BlockSpec((F, F), lambda i: (0, 0))],
        out_specs=pl.BlockSpec((RB, F), lambda i: (i, 0)),
        out_shape=jax.ShapeDtypeStruct((N, F), jnp.float32),
    )(acc, W)


def _log_softmax(acc):
    return pl.pallas_call(
        _lsm_body,
        grid=(N // RB,),
        in_specs=[pl.BlockSpec((NC, RB, F), lambda i: (0, i, 0))],
        out_specs=pl.BlockSpec((RB, F), lambda i: (i, 0)),
        out_shape=jax.ShapeDtypeStruct((N, F), jnp.float32),
    )(acc)


def _sc_spmm(sup, src3, dst3, zeros):
    mesh = plsc.VectorSubcoreMesh(core_axis_name="c", subcore_axis_name="s")

    @functools.partial(
        pl.kernel,
        out_type=jax.ShapeDtypeStruct((NC, N, F), jnp.float32),
        mesh=mesh,
        scratch_types=[
            pltpu.VMEM((2, G, CB), jnp.int32),
            pltpu.VMEM((2, G, CB), jnp.int32),
            pltpu.VMEM((NBUF, CB, F), jnp.float32),
            pltpu.VMEM_SHARED((N, F), jnp.float32),
            pltpu.SemaphoreType.DMA,
            pltpu.SemaphoreType.DMA,
            pltpu.SemaphoreType.DMA,
            pltpu.SemaphoreType.DMA,
            pltpu.SemaphoreType.DMA,
            pltpu.SemaphoreType.DMA,
        ],
    )
    def k(sup_hbm, src_hbm, dst_hbm, zeros_hbm, out_hbm,
          src_i, dst_i, rows_v, acc,
          gsem0, gsem1, gsem2, gsem3, isem0, isem1):
        cid = lax.axis_index("c")
        sid = lax.axis_index("s")
        wid = sid * NC + cid
        r0 = sid * RPS

        isems = (isem0, isem1)

        def idx_start(g, s):
            pltpu.async_copy(src_hbm.at[wid * NG + g], src_i.at[s], isems[s])
            pltpu.async_copy(dst_hbm.at[wid * NG + g], dst_i.at[s], isems[s])

        def idx_wait(g, s):
            pltpu.make_async_copy(src_hbm.at[wid * NG + g],
                                  src_i.at[s], isems[s]).wait()
            pltpu.make_async_copy(dst_hbm.at[wid * NG + g],
                                  dst_i.at[s], isems[s]).wait()

        idx_start(0, 0)
        idx_start(1, 1)
        pltpu.sync_copy(zeros_hbm.at[pl.ds(r0, RPS)], acc.at[pl.ds(r0, RPS)])

        @pl.when(sid == 0)
        def _():
            pltpu.sync_copy(zeros_hbm.at[pl.ds(TAIL0, TAILN)],
                            acc.at[pl.ds(TAIL0, TAILN)])

        plsc.subcore_barrier()

        BUFS = tuple(rows_v.at[b] for b in range(NBUF))
        GSEMS = (gsem0, gsem1, gsem2, gsem3)

        def g_start(s, jj, b):
            pltpu.async_copy(sup_hbm.at[src_i.at[s, jj]], BUFS[b], GSEMS[b])

        def g_wait(s, jj, b):
            pltpu.make_async_copy(sup_hbm.at[src_i.at[s, jj]],
                                  BUFS[b], GSEMS[b]).wait()

        def scat(s, jj, b):
            pltpu.sync_copy(BUFS[b], acc.at[dst_i.at[s, jj]], add=True)

        for g in range(NG):
            s = g % 2
            idx_wait(g, s)
            g_start(s, 0, 0)
            g_start(s, 1, 1)
            g_start(s, 2, 2)

            @pl.loop(0, NFULL, step=NBUF)
            def _(jj):
                for k in range(NBUF):
                    nxt = jj + k + NBUF - 1

                    @pl.when(nxt < G)
                    def _():
                        g_start(s, nxt, (k + NBUF - 1) % NBUF)

                    g_wait(s, jj + k, k)
                    scat(s, jj + k, k)

            for c in range(NFULL, G):
                g_wait(s, c, c % NBUF)
                scat(s, c, c % NBUF)
            if g + 2 < NG:
                idx_start(g + 2, s)

        plsc.subcore_barrier()
        pltpu.sync_copy(acc.at[pl.ds(r0, RPS)],
                        out_hbm.at[cid, pl.ds(r0, RPS)])

        @pl.when(sid == 0)
        def _():
            pltpu.sync_copy(acc.at[pl.ds(TAIL0, TAILN)],
                            out_hbm.at[cid, pl.ds(TAIL0, TAILN)])

    return k(sup, src3, dst3, zeros)


def kernel(x, edge_index, W1, W2):
    src3 = edge_index[0].astype(jnp.int32).reshape(NW * NG, G, CB)
    dst3 = edge_index[1].astype(jnp.int32).reshape(NW * NG, G, CB)
    zeros = jnp.zeros((N, F), jnp.float32)

    s1 = _matmul1(x, W1)
    a1 = _sc_spmm(s1, src3, dst3, zeros)
    s2 = _relu_matmul2(a1, W2)
    a2 = _sc_spmm(s2, src3, dst3, zeros)
    return _log_softmax(a2)


# speedup vs baseline: 1.2710x; 1.2710x over previous
import functools

import jax
import jax.numpy as jnp
from jax import lax
from jax.experimental import pallas as pl
from jax.experimental.pallas import tpu as pltpu
from jax.experimental.pallas import tpu_sc as plsc

N = 10000
F = 128
E = 320000
NC = 2
NS = 16
NW = NC * NS
EPW = E // NW
CB = 40
NCHUNK = EPW // CB
G = 50
NG = NCHUNK // G
NBUF = 4
NFULL = ((G - 1) // NBUF) * NBUF
RPS = 624
TAIL0 = RPS * NS
TAILN = N - TAIL0

RB = 1000


def _mm_relu_body(a_ref, w_ref, o_ref):
    s = a_ref[0] + a_ref[1]
    o_ref[...] = jnp.maximum(
        jnp.dot(s, w_ref[...], preferred_element_type=jnp.float32), 0.0)


def _mm_lsm_body(a_ref, w_ref, o_ref):
    s = jnp.dot(a_ref[0] + a_ref[1], w_ref[...],
                preferred_element_type=jnp.float32)
    m = jnp.max(s, axis=-1, keepdims=True)
    e = jnp.exp(s - m)
    o_ref[...] = s - m - jnp.log(jnp.sum(e, axis=-1, keepdims=True))


def _combine_mm(acc, W, body):
    return pl.pallas_call(
        body,
        grid=(N // RB,),
        in_specs=[pl.BlockSpec((NC, RB, F), lambda i: (0, i, 0)),
                  pl.BlockSpec((F, F), lambda i: (0, 0))],
        out_specs=pl.BlockSpec((RB, F), lambda i: (i, 0)),
        out_shape=jax.ShapeDtypeStruct((N, F), jnp.float32),
    )(acc, W)


def _sc_spmm(sup, src3, dst3, zeros):
    mesh = plsc.VectorSubcoreMesh(core_axis_name="c", subcore_axis_name="s")

    @functools.partial(
        pl.kernel,
        out_type=jax.ShapeDtypeStruct((NC, N, F), jnp.float32),
        mesh=mesh,
        scratch_types=[
            pltpu.VMEM((2, G, CB), jnp.int32),
            pltpu.VMEM((2, G, CB), jnp.int32),
            pltpu.VMEM((NBUF, CB, F), jnp.float32),
            pltpu.VMEM_SHARED((N, F), jnp.float32),
        ] + [pltpu.SemaphoreType.DMA] * (NBUF + 2),
    )
    def k(sup_hbm, src_hbm, dst_hbm, zeros_hbm, out_hbm,
          src_i, dst_i, rows_v, acc, *sems):
        cid = lax.axis_index("c")
        sid = lax.axis_index("s")
        wid = sid * NC + cid
        r0 = sid * RPS

        GSEMS = sems[:NBUF]
        isems = sems[NBUF:]

        def idx_start(g, s):
            pltpu.async_copy(src_hbm.at[wid * NG + g], src_i.at[s], isems[s])
            pltpu.async_copy(dst_hbm.at[wid * NG + g], dst_i.at[s], isems[s])

        def idx_wait(g, s):
            pltpu.make_async_copy(src_hbm.at[wid * NG + g],
                                  src_i.at[s], isems[s]).wait()
            pltpu.make_async_copy(dst_hbm.at[wid * NG + g],
                                  dst_i.at[s], isems[s]).wait()

        idx_start(0, 0)
        idx_start(1, 1)
        pltpu.sync_copy(zeros_hbm.at[pl.ds(r0, RPS)], acc.at[pl.ds(r0, RPS)])

        @pl.when(sid == 0)
        def _():
            pltpu.sync_copy(zeros_hbm.at[pl.ds(TAIL0, TAILN)],
                            acc.at[pl.ds(TAIL0, TAILN)])

        plsc.subcore_barrier()

        BUFS = tuple(rows_v.at[b] for b in range(NBUF))

        def g_start(s, jj, b):
            pltpu.async_copy(sup_hbm.at[src_i.at[s, jj]], BUFS[b], GSEMS[b])

        def g_wait(s, jj, b):
            pltpu.make_async_copy(sup_hbm.at[src_i.at[s, jj]],
                                  BUFS[b], GSEMS[b]).wait()

        def scat(s, jj, b):
            pltpu.sync_copy(BUFS[b], acc.at[dst_i.at[s, jj]], add=True)

        for g in range(NG):
            s = g % 2
            idx_wait(g, s)
            for c in range(NBUF - 1):
                g_start(s, c, c)

            @pl.loop(0, NFULL, step=NBUF)
            def _(jj):
                for k in range(NBUF):
                    nxt = jj + k + NBUF - 1

                    @pl.when(nxt < G)
                    def _():
                        g_start(s, nxt, (k + NBUF - 1) % NBUF)

                    g_wait(s, jj + k, k)
                    scat(s, jj + k, k)

            for c in range(NFULL, G):
                g_wait(s, c, c % NBUF)
                scat(s, c, c % NBUF)
            if g + 2 < NG:
                idx_start(g + 2, s)

        plsc.subcore_barrier()
        pltpu.sync_copy(acc.at[pl.ds(r0, RPS)],
                        out_hbm.at[cid, pl.ds(r0, RPS)])

        @pl.when(sid == 0)
        def _():
            pltpu.sync_copy(acc.at[pl.ds(TAIL0, TAILN)],
                            out_hbm.at[cid, pl.ds(TAIL0, TAILN)])

    return k(sup, src3, dst3, zeros)


def kernel(x, edge_index, W1, W2):
    src3 = edge_index[0].astype(jnp.int32).reshape(NW * NG, G, CB)
    dst3 = edge_index[1].astype(jnp.int32).reshape(NW * NG, G, CB)
    zeros = jnp.zeros((N, F), jnp.float32)

    a1 = _sc_spmm(x, src3, dst3, zeros)
    h = _combine_mm(a1, W1, _mm_relu_body)
    a2 = _sc_spmm(h, src3, dst3, zeros)
    return _combine_mm(a2, W2, _mm_lsm_body)


# speedup vs baseline: 1.3196x; 1.0382x over previous
import functools

import jax
import jax.numpy as jnp
from jax import lax
from jax.experimental import pallas as pl
from jax.experimental.pallas import tpu as pltpu
from jax.experimental.pallas import tpu_sc as plsc

N = 10000
F = 128
E = 320000
NC = 2
NS = 16
NW = NC * NS
EPW = E // NW
CB = 40
NCHUNK = EPW // CB
G = 50
NG = NCHUNK // G
NBUF = 4
NFULL = ((G - 1) // NBUF) * NBUF
RPS = 624
TAIL0 = RPS * NS
TAILN = N - TAIL0

RB = 1000


def _mm_relu_body(a_ref, w_ref, o_ref):
    s = a_ref[0] + a_ref[1]
    o_ref[...] = jnp.maximum(
        jnp.dot(s, w_ref[...], preferred_element_type=jnp.float32), 0.0)


def _mm_lsm_body(a_ref, w_ref, o_ref):
    s = jnp.dot(a_ref[0] + a_ref[1], w_ref[...],
                preferred_element_type=jnp.float32)
    m = jnp.max(s, axis=-1, keepdims=True)
    e = jnp.exp(s - m)
    o_ref[...] = s - m - jnp.log(jnp.sum(e, axis=-1, keepdims=True))


def _combine_mm(acc, W, body):
    return pl.pallas_call(
        body,
        grid=(N // RB,),
        in_specs=[pl.BlockSpec((NC, RB, F), lambda i: (0, i, 0)),
                  pl.BlockSpec((F, F), lambda i: (0, 0))],
        out_specs=pl.BlockSpec((RB, F), lambda i: (i, 0)),
        out_shape=jax.ShapeDtypeStruct((N, F), jnp.float32),
    )(acc, W)


def _sc_spmm(sup, src3, dst3):
    mesh = plsc.VectorSubcoreMesh(core_axis_name="c", subcore_axis_name="s")

    @functools.partial(
        pl.kernel,
        out_type=jax.ShapeDtypeStruct((NC, N, F), jnp.float32),
        mesh=mesh,
        scratch_types=[
            pltpu.VMEM((2, G, CB), jnp.int32),
            pltpu.VMEM((2, G, CB), jnp.int32),
            pltpu.VMEM((NBUF, CB, F), jnp.float32),
            pltpu.VMEM_SHARED((N, F), jnp.float32),
        ] + [pltpu.SemaphoreType.DMA] * (NBUF + 2),
    )
    def k(sup_hbm, src_hbm, dst_hbm, out_hbm,
          src_i, dst_i, rows_v, acc, *sems):
        cid = lax.axis_index("c")
        sid = lax.axis_index("s")
        wid = sid * NC + cid
        r0 = sid * RPS

        GSEMS = sems[:NBUF]
        isems = sems[NBUF:]

        def idx_start(g, s):
            pltpu.async_copy(src_hbm.at[wid * NG + g], src_i.at[s], isems[s])
            pltpu.async_copy(dst_hbm.at[wid * NG + g], dst_i.at[s], isems[s])

        def idx_wait(g, s):
            pltpu.make_async_copy(src_hbm.at[wid * NG + g],
                                  src_i.at[s], isems[s]).wait()
            pltpu.make_async_copy(dst_hbm.at[wid * NG + g],
                                  dst_i.at[s], isems[s]).wait()

        idx_start(0, 0)
        idx_start(1, 1)

        zv = jnp.zeros((16,), jnp.float32)

        @pl.loop(0, CB)
        def _(r):
            @pl.loop(0, F, step=16)
            def _(c2):
                rows_v[0, r, pl.ds(c2, 16)] = zv

        for i in range(RPS // CB):
            pltpu.sync_copy(rows_v.at[0],
                            acc.at[pl.ds(r0 + i * CB, CB)])
        _zrem = RPS % CB
        if _zrem:
            pltpu.sync_copy(rows_v.at[0, pl.ds(0, _zrem)],
                            acc.at[pl.ds(r0 + (RPS // CB) * CB, _zrem)])

        @pl.when(sid == 0)
        def _():
            pltpu.sync_copy(rows_v.at[0, pl.ds(0, TAILN)],
                            acc.at[pl.ds(TAIL0, TAILN)])

        plsc.subcore_barrier()

        BUFS = tuple(rows_v.at[b] for b in range(NBUF))

        def g_start(s, jj, b):
            pltpu.async_copy(sup_hbm.at[src_i.at[s, jj]], BUFS[b], GSEMS[b])

        def g_wait(s, jj, b):
            pltpu.make_async_copy(sup_hbm.at[src_i.at[s, jj]],
                                  BUFS[b], GSEMS[b]).wait()

        def scat(s, jj, b):
            pltpu.sync_copy(BUFS[b], acc.at[dst_i.at[s, jj]], add=True)

        for g in range(NG):
            s = g % 2
            idx_wait(g, s)
            for c in range(NBUF - 1):
                g_start(s, c, c)

            @pl.loop(0, NFULL, step=NBUF)
            def _(jj):
                for k in range(NBUF):
                    nxt = jj + k + NBUF - 1

                    @pl.when(nxt < G)
                    def _():
                        g_start(s, nxt, (k + NBUF - 1) % NBUF)

                    g_wait(s, jj + k, k)
                    scat(s, jj + k, k)

            for c in range(NFULL, G):
                g_wait(s, c, c % NBUF)
                scat(s, c, c % NBUF)
            if g + 2 < NG:
                idx_start(g + 2, s)

        plsc.subcore_barrier()
        pltpu.sync_copy(acc.at[pl.ds(r0, RPS)],
                        out_hbm.at[cid, pl.ds(r0, RPS)])

        @pl.when(sid == 0)
        def _():
            pltpu.sync_copy(acc.at[pl.ds(TAIL0, TAILN)],
                            out_hbm.at[cid, pl.ds(TAIL0, TAILN)])

    return k(sup, src3, dst3)


def kernel(x, edge_index, W1, W2):
    src3 = edge_index[0].astype(jnp.int32).reshape(NW * NG, G, CB)
    dst3 = edge_index[1].astype(jnp.int32).reshape(NW * NG, G, CB)
    a1 = _sc_spmm(x, src3, dst3)
    h = _combine_mm(a1, W1, _mm_relu_body)
    a2 = _sc_spmm(h, src3, dst3)
    return _combine_mm(a2, W2, _mm_lsm_body)


# speedup vs baseline: 1.3662x; 1.0353x over previous
import functools

import jax
import jax.numpy as jnp
from jax import lax
from jax.experimental import pallas as pl
from jax.experimental.pallas import tpu as pltpu
from jax.experimental.pallas import tpu_sc as plsc

N = 10000
F = 128
E = 320000
NC = 2
NS = 16
NW = NC * NS
EPW = E // NW
CB = 40
NCHUNK = EPW // CB
G = 50
NG = NCHUNK // G
NBUF = 4
NFULL = ((G - 1) // NBUF) * NBUF
RPS = 624
TAIL0 = RPS * NS
TAILN = N - TAIL0

RB = 1000


def _mm_relu_body(a_ref, w_ref, o_ref):
    s = a_ref[0] + a_ref[1]
    o_ref[...] = jnp.maximum(
        jnp.dot(s, w_ref[...], preferred_element_type=jnp.float32), 0.0)


def _mm_lsm_body(a_ref, w_ref, o_ref):
    s = jnp.dot(a_ref[0] + a_ref[1], w_ref[...],
                preferred_element_type=jnp.float32)
    m = jnp.max(s, axis=-1, keepdims=True)
    e = jnp.exp(s - m)
    o_ref[...] = s - m - jnp.log(jnp.sum(e, axis=-1, keepdims=True))


def _combine_mm(acc, W, body):
    return pl.pallas_call(
        body,
        grid=(N // RB,),
        in_specs=[pl.BlockSpec((NC, RB, F), lambda i: (0, i, 0)),
                  pl.BlockSpec((F, F), lambda i: (0, 0))],
        out_specs=pl.BlockSpec((RB, F), lambda i: (i, 0)),
        out_shape=jax.ShapeDtypeStruct((N, F), jnp.float32),
    )(acc, W)


def _sc_spmm(sup, ei4):
    mesh = plsc.VectorSubcoreMesh(core_axis_name="c", subcore_axis_name="s")

    @functools.partial(
        pl.kernel,
        out_type=jax.ShapeDtypeStruct((NC, N, F), jnp.float32),
        mesh=mesh,
        scratch_types=[
            pltpu.VMEM((2, G, CB), jnp.int32),
            pltpu.VMEM((2, G, CB), jnp.int32),
            pltpu.VMEM((NBUF, CB, F), jnp.float32),
            pltpu.VMEM_SHARED((N, F), jnp.float32),
        ] + [pltpu.SemaphoreType.DMA] * (NBUF + 2),
    )
    def k(sup_hbm, ei_hbm, out_hbm,
          src_i, dst_i, rows_v, acc, *sems):
        cid = lax.axis_index("c")
        sid = lax.axis_index("s")
        wid = sid * NC + cid
        r0 = sid * RPS

        GSEMS = sems[:NBUF]
        isems = sems[NBUF:]

        def idx_start(g, s):
            pltpu.async_copy(ei_hbm.at[0, wid * NG + g], src_i.at[s], isems[s])
            pltpu.async_copy(ei_hbm.at[1, wid * NG + g], dst_i.at[s], isems[s])

        def idx_wait(g, s):
            pltpu.make_async_copy(ei_hbm.at[0, wid * NG + g],
                                  src_i.at[s], isems[s]).wait()
            pltpu.make_async_copy(ei_hbm.at[1, wid * NG + g],
                                  dst_i.at[s], isems[s]).wait()

        idx_start(0, 0)
        idx_start(1, 1)

        zv = jnp.zeros((16,), jnp.float32)

        @pl.loop(0, CB)
        def _(r):
            @pl.loop(0, F, step=16)
            def _(c2):
                rows_v[0, r, pl.ds(c2, 16)] = zv

        for i in range(RPS // CB):
            pltpu.sync_copy(rows_v.at[0],
                            acc.at[pl.ds(r0 + i * CB, CB)])
        _zrem = RPS % CB
        if _zrem:
            pltpu.sync_copy(rows_v.at[0, pl.ds(0, _zrem)],
                            acc.at[pl.ds(r0 + (RPS // CB) * CB, _zrem)])

        @pl.when(sid == 0)
        def _():
            pltpu.sync_copy(rows_v.at[0, pl.ds(0, TAILN)],
                            acc.at[pl.ds(TAIL0, TAILN)])

        plsc.subcore_barrier()

        BUFS = tuple(rows_v.at[b] for b in range(NBUF))

        def g_start(s, jj, b):
            pltpu.async_copy(sup_hbm.at[src_i.at[s, jj]], BUFS[b], GSEMS[b])

        def g_wait(s, jj, b):
            pltpu.make_async_copy(sup_hbm.at[src_i.at[s, jj]],
                                  BUFS[b], GSEMS[b]).wait()

        def scat(s, jj, b):
            pltpu.sync_copy(BUFS[b], acc.at[dst_i.at[s, jj]], add=True)

        for g in range(NG):
            s = g % 2
            idx_wait(g, s)
            for c in range(NBUF - 1):
                g_start(s, c, c)

            @pl.loop(0, NFULL, step=NBUF)
            def _(jj):
                for k in range(NBUF):
                    nxt = jj + k + NBUF - 1

                    @pl.when(nxt < G)
                    def _():
                        g_start(s, nxt, (k + NBUF - 1) % NBUF)

                    g_wait(s, jj + k, k)
                    scat(s, jj + k, k)

            for c in range(NFULL, G):
                g_wait(s, c, c % NBUF)
                scat(s, c, c % NBUF)
            if g + 2 < NG:
                idx_start(g + 2, s)

        plsc.subcore_barrier()
        pltpu.sync_copy(acc.at[pl.ds(r0, RPS)],
                        out_hbm.at[cid, pl.ds(r0, RPS)])

        @pl.when(sid == 0)
        def _():
            pltpu.sync_copy(acc.at[pl.ds(TAIL0, TAILN)],
                            out_hbm.at[cid, pl.ds(TAIL0, TAILN)])

    return k(sup, ei4)


def kernel(x, edge_index, W1, W2):
    ei4 = edge_index.astype(jnp.int32).reshape(2, NW * NG, G, CB)
    a1 = _sc_spmm(x, ei4)
    h = _combine_mm(a1, W1, _mm_relu_body)
    a2 = _sc_spmm(h, ei4)
    return _combine_mm(a2, W2, _mm_lsm_body)
